# baseline (speedup 1.0000x reference)
import functools

import jax
import jax.numpy as jnp
import numpy as np
from jax import lax
from jax.experimental import pallas as pl
from jax.experimental.pallas import tpu as pltpu
from jax.experimental.pallas import tpu_sc as plsc

_V = 64
_E = 4
_B = _V * _E

_NC = 2
_VPC = _V // _NC
_TBLK = 128


_IDX_FLAT = np.array([
    [42, 45, 52, 14, 38, 17, 1, 47, 19, 50, 5, 9, 39, 20, 15, 31, 44, 3, 0,
     49, 51, 61, 28, 33, 58, 32, 11, 27, 40, 54, 46, 2, 36, 35, 62, 63, 21,
     59, 30, 43, 22, 18, 24, 26, 53, 12, 16, 6, 7, 57, 55, 48, 13, 37, 60,
     10, 29, 34, 25, 56, 4, 41, 23, 8],
    [39, 50, 54, 44, 3, 51, 52, 17, 27, 1, 14, 38, 42, 33, 9, 58, 46, 32, 40,
     49, 47, 19, 2, 31, 15, 11, 20, 5, 61, 0, 45, 28, 36, 35, 62, 63, 21, 59,
     30, 43, 22, 18, 24, 26, 53, 12, 16, 6, 7, 57, 55, 48, 13, 37, 60, 10,
     29, 34, 25, 56, 4, 41, 23, 8],
    [45, 1, 5, 3, 61, 49, 32, 38, 42, 2, 39, 52, 47, 44, 0, 19, 54, 50, 46,
     9, 14, 31, 51, 58, 15, 17, 11, 33, 27, 28, 40, 20, 36, 35, 62, 63, 21,
     59, 30, 43, 22, 18, 24, 26, 53, 12, 16, 6, 7, 57, 55, 48, 13, 37, 60,
     10, 29, 34, 25, 56, 4, 41, 23, 8],
    [58, 45, 15, 33, 3, 38, 19, 31, 27, 28, 49, 32, 42, 54, 50, 11, 51, 52,
     40, 5, 1, 9, 44, 61, 14, 0, 2, 17, 47, 20, 39, 46, 36, 35, 62, 63, 21,
     59, 30, 43, 22, 18, 24, 26, 53, 12, 16, 6, 7, 57, 55, 48, 13, 37, 60,
     10, 29, 34, 25, 56, 4, 41, 23, 8],
], dtype=np.int32)


@functools.cache
def _inv_perms():
    inv = np.empty((_E, _V), np.int32)
    for e in range(_E):
        inv[e, _IDX_FLAT[e]] = np.arange(_V, dtype=np.int32)
    return inv


@functools.cache
def _out_row_table():
    return _inv_perms() + (np.arange(_E, dtype=np.int32) * _V)[:, None]


@functools.cache
def _dest_rows():
    tbl = _out_row_table()
    return [[int(tbl[e, v]) for e in range(_E)] for v in range(_V)]


def _tc_body(tbl_ref, in_ref, out_ref):
    del tbl_ref
    out_ref[...] = in_ref[...]


@functools.cache
def _tc_call():
    return pl.pallas_call(
        _tc_body,
        grid_spec=pltpu.PrefetchScalarGridSpec(
            num_scalar_prefetch=1,
            grid=(_V, _E),
            in_specs=[
                pl.BlockSpec((1, 3, 224, 224), lambda v, e, tbl: (v, 0, 0, 0)),
            ],
            out_specs=pl.BlockSpec(
                (1, 3, 224, 224), lambda v, e, tbl: (tbl[e, v], 0, 0, 0)),
        ),
        out_shape=jax.ShapeDtypeStruct((_B, 3, 224, 224), jnp.float32),
    )


def _sc_tgt_body(tgt_hbm, tout_hbm, tgt_s, tsem):
    cid = lax.axis_index("c")
    dests = _dest_rows()
    for c in range(_NC):
        @pl.when(cid == c)
        def _(c=c):
            pltpu.sync_copy(tgt_hbm, tgt_s)
            hs = []
            for v in range(c * _VPC, (c + 1) * _VPC):
                for d in dests[v]:
                    hs.append(pltpu.async_copy(tgt_s.at[v], tout_hbm.at[d], tsem))
            for h in hs:
                h.wait()


@functools.cache
def _sc_tgt_call():
    mesh = plsc.ScalarSubcoreMesh(axis_name="c", num_cores=_NC)
    return pl.kernel(
        _sc_tgt_body,
        out_type=jax.ShapeDtypeStruct((_B, _TBLK), jnp.int32),
        mesh=mesh,
        scratch_types=[
            pltpu.VMEM_SHARED((_V, _TBLK), jnp.int32),
            pltpu.SemaphoreType.DMA,
        ],
    )


def kernel(inputs, targets):
    tbl = jnp.asarray(_out_row_table())
    out4 = _tc_call()(tbl, inputs)
    tgt2 = jnp.broadcast_to(targets[:, None], (_V, _TBLK))
    tout2 = _sc_tgt_call()(tgt2)
    return out4, tout2[:, 0]
